# baseline (speedup 1.0000x reference)
import functools

import jax
import jax.numpy as jnp
from jax import lax
from jax.experimental import pallas as pl
from jax.experimental.pallas import tpu as pltpu
from jax.experimental.pallas import tpu_sc as plsc

_N = 10000
_E = 320000
_D = 128
_NC, _NS = 2, 16
_NW = _NC * _NS
_NPAD = 10240
_CHUNK = 128
_NCHUNK = 2560
_BLK = 16
_CPT = _NCHUNK // _NS
_EPAD = _NCHUNK * _CHUNK
_EPT = _EPAD // _NW
_RPT = _NPAD // _NS
_BM = 1024


def _deg_body(dst_hbm, out_hbm, dst_v, deg_v):
    cid = lax.axis_index("c")
    sid = lax.axis_index("s")
    wid = sid * _NC + cid
    pltpu.sync_copy(dst_hbm.at[wid], dst_v)

    def zero(i, carry):
        deg_v[i, :] = jnp.zeros((_NS,), jnp.float32)
        return carry

    lax.fori_loop(0, _RPT, zero, 0)

    ones = jnp.ones((16,), jnp.float32)

    def accum(i, carry):
        idx = dst_v[pl.ds(i * 16, 16)]
        plsc.addupdate_scatter(deg_v, [idx >> 4, idx & 15], ones)
        return carry

    lax.fori_loop(0, _EPT // 16, accum, 0)
    pltpu.sync_copy(deg_v, out_hbm.at[wid])


def _scatter_body(
    y_hbm, src_hbm, dst_hbm, out_hbm, src_ring, dst_ring, buf0, buf1, acc, sem0, sem1
):
    cid = lax.axis_index("c")
    sid = lax.axis_index("s")

    @pl.when(cid == 0)
    def _core0():
        base = sid * _CPT
        rows = pl.ds(sid * _RPT, _RPT)
        pltpu.sync_copy(y_hbm.at[rows], acc.at[rows])
        plsc.subcore_barrier()

        pltpu.sync_copy(src_hbm.at[pl.ds(base, _BLK)], src_ring.at[0])
        pltpu.sync_copy(dst_hbm.at[pl.ds(base, _BLK)], dst_ring.at[0])
        pltpu.async_copy(y_hbm.at[src_ring.at[0, 0]], buf0, sem0)

        def body(g, carry):
            c0 = 2 * g
            c1 = c0 + 1
            c2 = c0 + 2

            @pl.when(jnp.logical_and(c0 % _BLK == 0, c0 + _BLK < _CPT))
            def _():
                b1 = c0 // _BLK + 1
                pltpu.sync_copy(
                    src_hbm.at[pl.ds(base + b1 * _BLK, _BLK)], src_ring.at[b1 % 2]
                )
                pltpu.sync_copy(
                    dst_hbm.at[pl.ds(base + b1 * _BLK, _BLK)], dst_ring.at[b1 % 2]
                )

            h1 = pltpu.async_copy(
                y_hbm.at[src_ring.at[(c1 // _BLK) % 2, c1 % _BLK]], buf1, sem1
            )
            pltpu.make_async_copy(
                y_hbm.at[src_ring.at[(c0 // _BLK) % 2, c0 % _BLK]], buf0, sem0
            ).wait()
            pltpu.sync_copy(
                buf0, acc.at[dst_ring.at[(c0 // _BLK) % 2, c0 % _BLK]], add=True
            )

            @pl.when(c2 < _CPT)
            def _():
                pltpu.async_copy(
                    y_hbm.at[src_ring.at[(c2 // _BLK) % 2, c2 % _BLK]], buf0, sem0
                )

            h1.wait()
            pltpu.sync_copy(
                buf1, acc.at[dst_ring.at[(c1 // _BLK) % 2, c1 % _BLK]], add=True
            )
            return carry

        lax.fori_loop(0, _CPT // 2, body, 0)
        plsc.subcore_barrier()
        pltpu.sync_copy(acc.at[rows], out_hbm.at[rows])


@functools.cache
def _get_sc_kernels():
    mesh = plsc.VectorSubcoreMesh(
        core_axis_name="c", subcore_axis_name="s", num_cores=_NC, num_subcores=_NS
    )
    deg = pl.kernel(
        _deg_body,
        out_type=jax.ShapeDtypeStruct((_NW, _RPT, _NS), jnp.float32),
        mesh=mesh,
        scratch_types=[
            pltpu.VMEM((_EPT,), jnp.int32),
            pltpu.VMEM((_RPT, _NS), jnp.float32),
        ],
        compiler_params=pltpu.CompilerParams(needs_layout_passes=False),
    )
    scat = pl.kernel(
        _scatter_body,
        out_type=jax.ShapeDtypeStruct((_NPAD, _D), jnp.float32),
        mesh=mesh,
        scratch_types=[
            pltpu.VMEM((2, _BLK, _CHUNK), jnp.int32),
            pltpu.VMEM((2, _BLK, _CHUNK), jnp.int32),
            pltpu.VMEM((_CHUNK, _D), jnp.float32),
            pltpu.VMEM((_CHUNK, _D), jnp.float32),
            pltpu.VMEM_SHARED((_NPAD, _D), jnp.float32),
            pltpu.SemaphoreType.DMA,
            pltpu.SemaphoreType.DMA,
        ],
    )
    return deg, scat


def _dinv_block(deg_ref):
    deg = jnp.sum(deg_ref[...], axis=0) + 1.0
    return lax.rsqrt(deg)


def _t1_body(x_ref, w_ref, deg_ref, o_ref):
    dinv = _dinv_block(deg_ref)
    xw = jnp.dot(x_ref[...], w_ref[...], preferred_element_type=jnp.float32)
    o_ref[...] = xw * dinv[:, None]


def _t2_body(a_ref, deg_ref, b_ref, w_ref, o_ref):
    i = pl.program_id(0)
    dinv = _dinv_block(deg_ref)
    pre = a_ref[...] * dinv[:, None] + b_ref[...]
    h = jnp.maximum(pre, 0.0)
    y2 = jnp.dot(h, w_ref[...], preferred_element_type=jnp.float32) * dinv[:, None]
    rows = i * _BM + lax.broadcasted_iota(jnp.int32, (_BM, 1), 0)
    o_ref[...] = jnp.where(rows < _N, y2, 0.0)


def _t3_body(a_ref, deg_ref, b_ref, o_ref, lp_ref):
    dinv = _dinv_block(deg_ref)
    out = a_ref[...] * dinv[:, None] + b_ref[...]
    o_ref[...] = out
    m = jnp.max(out, axis=1, keepdims=True)
    ex = jnp.exp(out - m)
    lse = jnp.log(jnp.sum(ex, axis=1, keepdims=True)) + m
    lp_ref[...] = out - lse


_GRID = (_NPAD // _BM,)
_row_spec = pl.BlockSpec((_BM, _D), lambda i: (i, 0))
_acc_spec = pl.BlockSpec((_NC, _BM, _D), lambda i: (0, i, 0))
_deg_spec = pl.BlockSpec((_NW, _BM), lambda i: (0, i))
_w_spec = pl.BlockSpec((_D, _D), lambda i: (0, 0))
_b_spec = pl.BlockSpec((1, _D), lambda i: (0, 0))
_f32 = jnp.float32

_t1 = pl.pallas_call(
    _t1_body,
    grid=_GRID,
    in_specs=[_row_spec, _w_spec, _deg_spec],
    out_specs=_row_spec,
    out_shape=jax.ShapeDtypeStruct((_NPAD, _D), _f32),
)

_t2 = pl.pallas_call(
    _t2_body,
    grid=_GRID,
    in_specs=[_row_spec, _deg_spec, _b_spec, _w_spec],
    out_specs=_row_spec,
    out_shape=jax.ShapeDtypeStruct((_NPAD, _D), _f32),
)

_t3 = pl.pallas_call(
    _t3_body,
    grid=_GRID,
    in_specs=[_row_spec, _deg_spec, _b_spec],
    out_specs=[_row_spec, _row_spec],
    out_shape=[
        jax.ShapeDtypeStruct((_NPAD, _D), _f32),
        jax.ShapeDtypeStruct((_NPAD, _D), _f32),
    ],
)


def kernel(x, edge_index, W1, b1, W2, b2):
    src = edge_index[0].astype(jnp.int32)
    dst = edge_index[1].astype(jnp.int32)
    pad = jnp.full((_EPAD - _E,), _N, jnp.int32)
    src_p = jnp.concatenate([src, pad]).reshape(_NCHUNK, _CHUNK)
    dst_p = jnp.concatenate([dst, pad]).reshape(_NCHUNK, _CHUNK)
    dst_flat = dst_p.reshape(_NW, _EPT)
    x_p = jnp.pad(x, ((0, _NPAD - _N), (0, 0)))
    b1r = b1.reshape(1, _D)
    b2r = b2.reshape(1, _D)

    deg_kernel, scatter_kernel = _get_sc_kernels()
    degp = deg_kernel(dst_flat).reshape(_NW, _NPAD)
    y1 = _t1(x_p, W1, degp)
    acc1 = scatter_kernel(y1, src_p, dst_p)
    y2 = _t2(acc1, degp, b1r, W2)
    acc2 = scatter_kernel(y2, src_p, dst_p)
    out, logp = _t3(acc2, degp, b2r)
    return out[:_N], logp[:_N]


# speedup vs baseline: 1.2527x; 1.2527x over previous
import functools

import jax
import jax.numpy as jnp
from jax import lax
from jax.experimental import pallas as pl
from jax.experimental.pallas import tpu as pltpu
from jax.experimental.pallas import tpu_sc as plsc

_N = 10000
_E = 320000
_D = 128
_NC, _NS = 2, 16
_NW = _NC * _NS
_NPAD = 10240
_CHUNK = 128
_NCHUNK = 2560
_BLK = 16
_CPT0 = 128
_CPT1 = _NCHUNK // _NS - _CPT0
_EPAD = _NCHUNK * _CHUNK
_EPT = _EPAD // _NW
_RPT = _NPAD // _NS
_BM = 1024


def _deg_body(dst_hbm, out_hbm, dst_v, deg_v):
    cid = lax.axis_index("c")
    sid = lax.axis_index("s")
    wid = sid * _NC + cid
    pltpu.sync_copy(dst_hbm.at[wid], dst_v)

    def zero(i, carry):
        deg_v[i, :] = jnp.zeros((_NS,), jnp.float32)
        return carry

    lax.fori_loop(0, _RPT, zero, 0)

    ones = jnp.ones((16,), jnp.float32)

    def accum(i, carry):
        idx = dst_v[pl.ds(i * 16, 16)]
        plsc.addupdate_scatter(deg_v, [idx >> 4, idx & 15], ones)
        return carry

    lax.fori_loop(0, _EPT // 16, accum, 0)
    pltpu.sync_copy(deg_v, out_hbm.at[wid])


def _scatter_body(
    y_hbm, src_hbm, dst_hbm, out_hbm, src_ring, dst_ring, buf0, buf1, acc, sem0, sem1
):
    cid = lax.axis_index("c")
    sid = lax.axis_index("s")
    cpt = jnp.where(cid == 0, _CPT0, _CPT1)
    base = jnp.where(cid == 0, sid * _CPT0, _NS * _CPT0 + sid * _CPT1)

    zeros16 = jnp.zeros((16,), jnp.float32)

    def zrow(k, carry):
        buf0[k // 8, pl.ds((k % 8) * 16, 16)] = zeros16
        return carry

    lax.fori_loop(0, _CHUNK * 8, zrow, 0)
    for r in range(_RPT // _CHUNK):
        pltpu.sync_copy(buf0, acc.at[pl.ds(sid * _RPT + r * _CHUNK, _CHUNK)])

    pltpu.sync_copy(src_hbm.at[pl.ds(base, _BLK)], src_ring.at[0])
    pltpu.sync_copy(dst_hbm.at[pl.ds(base, _BLK)], dst_ring.at[0])
    pltpu.async_copy(y_hbm.at[src_ring.at[0, 0]], buf0, sem0)
    plsc.subcore_barrier()

    def body(g, carry):
        c0 = 2 * g
        c1 = c0 + 1
        c2 = c0 + 2

        @pl.when(jnp.logical_and(c0 % _BLK == 0, c0 + _BLK < cpt))
        def _():
            b1 = c0 // _BLK + 1
            pltpu.sync_copy(
                src_hbm.at[pl.ds(base + b1 * _BLK, _BLK)], src_ring.at[b1 % 2]
            )
            pltpu.sync_copy(
                dst_hbm.at[pl.ds(base + b1 * _BLK, _BLK)], dst_ring.at[b1 % 2]
            )

        h1 = pltpu.async_copy(
            y_hbm.at[src_ring.at[(c1 // _BLK) % 2, c1 % _BLK]], buf1, sem1
        )
        pltpu.make_async_copy(
            y_hbm.at[src_ring.at[(c0 // _BLK) % 2, c0 % _BLK]], buf0, sem0
        ).wait()
        pltpu.sync_copy(
            buf0, acc.at[dst_ring.at[(c0 // _BLK) % 2, c0 % _BLK]], add=True
        )

        @pl.when(c2 < cpt)
        def _():
            pltpu.async_copy(
                y_hbm.at[src_ring.at[(c2 // _BLK) % 2, c2 % _BLK]], buf0, sem0
            )

        h1.wait()
        pltpu.sync_copy(
            buf1, acc.at[dst_ring.at[(c1 // _BLK) % 2, c1 % _BLK]], add=True
        )
        return carry

    lax.fori_loop(0, cpt // 2, body, 0)
    plsc.subcore_barrier()
    pltpu.sync_copy(
        acc.at[pl.ds(sid * _RPT, _RPT)], out_hbm.at[cid, pl.ds(sid * _RPT, _RPT)]
    )


@functools.cache
def _get_sc_kernels():
    mesh = plsc.VectorSubcoreMesh(
        core_axis_name="c", subcore_axis_name="s", num_cores=_NC, num_subcores=_NS
    )
    deg = pl.kernel(
        _deg_body,
        out_type=jax.ShapeDtypeStruct((_NW, _RPT, _NS), jnp.float32),
        mesh=mesh,
        scratch_types=[
            pltpu.VMEM((_EPT,), jnp.int32),
            pltpu.VMEM((_RPT, _NS), jnp.float32),
        ],
        compiler_params=pltpu.CompilerParams(needs_layout_passes=False),
    )
    scat = pl.kernel(
        _scatter_body,
        out_type=jax.ShapeDtypeStruct((_NC, _NPAD, _D), jnp.float32),
        mesh=mesh,
        scratch_types=[
            pltpu.VMEM((2, _BLK, _CHUNK), jnp.int32),
            pltpu.VMEM((2, _BLK, _CHUNK), jnp.int32),
            pltpu.VMEM((_CHUNK, _D), jnp.float32),
            pltpu.VMEM((_CHUNK, _D), jnp.float32),
            pltpu.VMEM_SHARED((_NPAD, _D), jnp.float32),
            pltpu.SemaphoreType.DMA,
            pltpu.SemaphoreType.DMA,
        ],
    )
    return deg, scat


def _dinv_block(deg_ref):
    deg = jnp.sum(deg_ref[...], axis=0) + 1.0
    return lax.rsqrt(deg)


def _t1_body(x_ref, w_ref, deg_ref, o_ref):
    dinv = _dinv_block(deg_ref)
    xw = jnp.dot(x_ref[...], w_ref[...], preferred_element_type=jnp.float32)
    o_ref[...] = xw * dinv[:, None]


def _t2_body(a_ref, y_ref, deg_ref, b_ref, w_ref, o_ref):
    i = pl.program_id(0)
    dinv = _dinv_block(deg_ref)
    pre = (a_ref[0] + a_ref[1] + y_ref[...]) * dinv[:, None] + b_ref[...]
    h = jnp.maximum(pre, 0.0)
    y2 = jnp.dot(h, w_ref[...], preferred_element_type=jnp.float32) * dinv[:, None]
    rows = i * _BM + lax.broadcasted_iota(jnp.int32, (_BM, 1), 0)
    o_ref[...] = jnp.where(rows < _N, y2, 0.0)


def _t3_body(a_ref, y_ref, deg_ref, b_ref, o_ref, lp_ref):
    dinv = _dinv_block(deg_ref)
    out = (a_ref[0] + a_ref[1] + y_ref[...]) * dinv[:, None] + b_ref[...]
    o_ref[...] = out
    m = jnp.max(out, axis=1, keepdims=True)
    ex = jnp.exp(out - m)
    lse = jnp.log(jnp.sum(ex, axis=1, keepdims=True)) + m
    lp_ref[...] = out - lse


_GRID = (_NPAD // _BM,)
_row_spec = pl.BlockSpec((_BM, _D), lambda i: (i, 0))
_acc_spec = pl.BlockSpec((_NC, _BM, _D), lambda i: (0, i, 0))
_deg_spec = pl.BlockSpec((_NW, _BM), lambda i: (0, i))
_w_spec = pl.BlockSpec((_D, _D), lambda i: (0, 0))
_b_spec = pl.BlockSpec((1, _D), lambda i: (0, 0))
_f32 = jnp.float32

_t1 = pl.pallas_call(
    _t1_body,
    grid=_GRID,
    in_specs=[_row_spec, _w_spec, _deg_spec],
    out_specs=_row_spec,
    out_shape=jax.ShapeDtypeStruct((_NPAD, _D), _f32),
)

_t2 = pl.pallas_call(
    _t2_body,
    grid=_GRID,
    in_specs=[_acc_spec, _row_spec, _deg_spec, _b_spec, _w_spec],
    out_specs=_row_spec,
    out_shape=jax.ShapeDtypeStruct((_NPAD, _D), _f32),
)

_t3 = pl.pallas_call(
    _t3_body,
    grid=_GRID,
    in_specs=[_acc_spec, _row_spec, _deg_spec, _b_spec],
    out_specs=[_row_spec, _row_spec],
    out_shape=[
        jax.ShapeDtypeStruct((_NPAD, _D), _f32),
        jax.ShapeDtypeStruct((_NPAD, _D), _f32),
    ],
)


def kernel(x, edge_index, W1, b1, W2, b2):
    src = edge_index[0].astype(jnp.int32)
    dst = edge_index[1].astype(jnp.int32)
    pad = jnp.full((_EPAD - _E,), _N, jnp.int32)
    src_p = jnp.concatenate([src, pad]).reshape(_NCHUNK, _CHUNK)
    dst_p = jnp.concatenate([dst, pad]).reshape(_NCHUNK, _CHUNK)
    dst_flat = dst_p.reshape(_NW, _EPT)
    x_p = jnp.pad(x, ((0, _NPAD - _N), (0, 0)))
    b1r = b1.reshape(1, _D)
    b2r = b2.reshape(1, _D)

    deg_kernel, scatter_kernel = _get_sc_kernels()
    degp = deg_kernel(dst_flat).reshape(_NW, _NPAD)
    y1 = _t1(x_p, W1, degp)
    acc1 = scatter_kernel(y1, src_p, dst_p)
    y2 = _t2(acc1, y1, degp, b1r, W2)
    acc2 = scatter_kernel(y2, src_p, dst_p)
    out, logp = _t3(acc2, y2, degp, b2r)
    return out[:_N], logp[:_N]


# speedup vs baseline: 1.2532x; 1.0005x over previous
import functools

import jax
import jax.numpy as jnp
from jax import lax
from jax.experimental import pallas as pl
from jax.experimental.pallas import tpu as pltpu
from jax.experimental.pallas import tpu_sc as plsc

_N = 10000
_E = 320000
_D = 128
_NC, _NS = 2, 16
_NW = _NC * _NS
_NPAD = 10240
_CHUNK = 128
_NCHUNK = 2560
_BLK = 16
_CPT0 = 128
_CPT1 = _NCHUNK // _NS - _CPT0
_EPAD = _NCHUNK * _CHUNK
_EPT = _EPAD // _NW
_RPT = _NPAD // _NS
_NSUB = 2
_BM = 1024


def _deg_body(dst_hbm, out_hbm, dst_v, deg_v):
    cid = lax.axis_index("c")
    sid = lax.axis_index("s")
    wid = sid * _NC + cid
    pltpu.sync_copy(dst_hbm.at[wid], dst_v)

    def zero(i, carry):
        deg_v[i, :] = jnp.zeros((_NS,), jnp.float32)
        return carry

    lax.fori_loop(0, _RPT, zero, 0)

    ones = jnp.ones((16,), jnp.float32)

    def accum(i, carry):
        idx = dst_v[pl.ds(i * 16, 16)]
        plsc.addupdate_scatter(deg_v, [idx >> 4, idx & 15], ones)
        return carry

    lax.fori_loop(0, _EPT // 16, accum, 0)
    pltpu.sync_copy(deg_v, out_hbm.at[wid])


def _scatter_body(
    y_hbm, src_hbm, dst_hbm, out_hbm, src_ring, dst_ring, buf0, buf1, acc, sem0, sem1
):
    cid = lax.axis_index("c")
    sid = lax.axis_index("s")
    cpt = jnp.where(cid == 0, _CPT0, _CPT1)
    base = jnp.where(cid == 0, sid * _CPT0, _NS * _CPT0 + sid * _CPT1)

    zeros16 = jnp.zeros((16,), jnp.float32)

    def zrow(k, carry):
        buf0[k // 8, pl.ds((k % 8) * 16, 16)] = zeros16
        return carry

    lax.fori_loop(0, _CHUNK * 8, zrow, 0)
    for r in range(_RPT // _CHUNK):
        pltpu.sync_copy(buf0, acc.at[pl.ds(sid * _RPT + r * _CHUNK, _CHUNK)])

    _SUB = _CHUNK // _NSUB

    def gather_start(c, buf, sem):
        h = (c // _BLK) % 2
        s = c % _BLK
        for k in range(_NSUB):
            pltpu.async_copy(
                y_hbm.at[src_ring.at[h, s, pl.ds(k * _SUB, _SUB)]],
                buf.at[pl.ds(k * _SUB, _SUB)],
                sem,
            )

    def gather_wait(c, buf, sem):
        h = (c // _BLK) % 2
        s = c % _BLK
        for k in range(_NSUB):
            pltpu.make_async_copy(
                y_hbm.at[src_ring.at[h, s, pl.ds(k * _SUB, _SUB)]],
                buf.at[pl.ds(k * _SUB, _SUB)],
                sem,
            ).wait()

    pltpu.sync_copy(src_hbm.at[pl.ds(base, _BLK)], src_ring.at[0])
    pltpu.sync_copy(dst_hbm.at[pl.ds(base, _BLK)], dst_ring.at[0])
    gather_start(0, buf0, sem0)
    plsc.subcore_barrier()

    def body(g, carry):
        c0 = 2 * g
        c1 = c0 + 1
        c2 = c0 + 2

        @pl.when(jnp.logical_and(c0 % _BLK == 0, c0 + _BLK < cpt))
        def _():
            b1 = c0 // _BLK + 1
            pltpu.sync_copy(
                src_hbm.at[pl.ds(base + b1 * _BLK, _BLK)], src_ring.at[b1 % 2]
            )
            pltpu.sync_copy(
                dst_hbm.at[pl.ds(base + b1 * _BLK, _BLK)], dst_ring.at[b1 % 2]
            )

        gather_start(c1, buf1, sem1)
        gather_wait(c0, buf0, sem0)
        pltpu.sync_copy(
            buf0, acc.at[dst_ring.at[(c0 // _BLK) % 2, c0 % _BLK]], add=True
        )

        @pl.when(c2 < cpt)
        def _():
            gather_start(c2, buf0, sem0)

        gather_wait(c1, buf1, sem1)
        pltpu.sync_copy(
            buf1, acc.at[dst_ring.at[(c1 // _BLK) % 2, c1 % _BLK]], add=True
        )
        return carry

    lax.fori_loop(0, cpt // 2, body, 0)
    plsc.subcore_barrier()
    pltpu.sync_copy(
        acc.at[pl.ds(sid * _RPT, _RPT)], out_hbm.at[cid, pl.ds(sid * _RPT, _RPT)]
    )


@functools.cache
def _get_sc_kernels():
    mesh = plsc.VectorSubcoreMesh(
        core_axis_name="c", subcore_axis_name="s", num_cores=_NC, num_subcores=_NS
    )
    deg = pl.kernel(
        _deg_body,
        out_type=jax.ShapeDtypeStruct((_NW, _RPT, _NS), jnp.float32),
        mesh=mesh,
        scratch_types=[
            pltpu.VMEM((_EPT,), jnp.int32),
            pltpu.VMEM((_RPT, _NS), jnp.float32),
        ],
        compiler_params=pltpu.CompilerParams(needs_layout_passes=False),
    )
    scat = pl.kernel(
        _scatter_body,
        out_type=jax.ShapeDtypeStruct((_NC, _NPAD, _D), jnp.float32),
        mesh=mesh,
        scratch_types=[
            pltpu.VMEM((2, _BLK, _CHUNK), jnp.int32),
            pltpu.VMEM((2, _BLK, _CHUNK), jnp.int32),
            pltpu.VMEM((_CHUNK, _D), jnp.float32),
            pltpu.VMEM((_CHUNK, _D), jnp.float32),
            pltpu.VMEM_SHARED((_NPAD, _D), jnp.float32),
            pltpu.SemaphoreType.DMA,
            pltpu.SemaphoreType.DMA,
        ],
    )
    return deg, scat


def _dinv_block(deg_ref):
    deg = jnp.sum(deg_ref[...], axis=0) + 1.0
    return lax.rsqrt(deg)


def _t1_body(x_ref, w_ref, deg_ref, o_ref):
    dinv = _dinv_block(deg_ref)
    xw = jnp.dot(x_ref[...], w_ref[...], preferred_element_type=jnp.float32)
    o_ref[...] = xw * dinv[:, None]


def _t2_body(a_ref, y_ref, deg_ref, b_ref, w_ref, o_ref):
    i = pl.program_id(0)
    dinv = _dinv_block(deg_ref)
    pre = (a_ref[0] + a_ref[1] + y_ref[...]) * dinv[:, None] + b_ref[...]
    h = jnp.maximum(pre, 0.0)
    y2 = jnp.dot(h, w_ref[...], preferred_element_type=jnp.float32) * dinv[:, None]
    rows = i * _BM + lax.broadcasted_iota(jnp.int32, (_BM, 1), 0)
    o_ref[...] = jnp.where(rows < _N, y2, 0.0)


def _t3_body(a_ref, y_ref, deg_ref, b_ref, o_ref, lp_ref):
    dinv = _dinv_block(deg_ref)
    out = (a_ref[0] + a_ref[1] + y_ref[...]) * dinv[:, None] + b_ref[...]
    o_ref[...] = out
    m = jnp.max(out, axis=1, keepdims=True)
    ex = jnp.exp(out - m)
    lse = jnp.log(jnp.sum(ex, axis=1, keepdims=True)) + m
    lp_ref[...] = out - lse


_GRID = (_NPAD // _BM,)
_row_spec = pl.BlockSpec((_BM, _D), lambda i: (i, 0))
_acc_spec = pl.BlockSpec((_NC, _BM, _D), lambda i: (0, i, 0))
_deg_spec = pl.BlockSpec((_NW, _BM), lambda i: (0, i))
_w_spec = pl.BlockSpec((_D, _D), lambda i: (0, 0))
_b_spec = pl.BlockSpec((1, _D), lambda i: (0, 0))
_f32 = jnp.float32

_t1 = pl.pallas_call(
    _t1_body,
    grid=_GRID,
    in_specs=[_row_spec, _w_spec, _deg_spec],
    out_specs=_row_spec,
    out_shape=jax.ShapeDtypeStruct((_NPAD, _D), _f32),
)

_t2 = pl.pallas_call(
    _t2_body,
    grid=_GRID,
    in_specs=[_acc_spec, _row_spec, _deg_spec, _b_spec, _w_spec],
    out_specs=_row_spec,
    out_shape=jax.ShapeDtypeStruct((_NPAD, _D), _f32),
)

_t3 = pl.pallas_call(
    _t3_body,
    grid=_GRID,
    in_specs=[_acc_spec, _row_spec, _deg_spec, _b_spec],
    out_specs=[_row_spec, _row_spec],
    out_shape=[
        jax.ShapeDtypeStruct((_NPAD, _D), _f32),
        jax.ShapeDtypeStruct((_NPAD, _D), _f32),
    ],
)


def kernel(x, edge_index, W1, b1, W2, b2):
    src = edge_index[0].astype(jnp.int32)
    dst = edge_index[1].astype(jnp.int32)
    pad = jnp.full((_EPAD - _E,), _N, jnp.int32)
    src_p = jnp.concatenate([src, pad]).reshape(_NCHUNK, _CHUNK)
    dst_p = jnp.concatenate([dst, pad]).reshape(_NCHUNK, _CHUNK)
    dst_flat = dst_p.reshape(_NW, _EPT)
    x_p = jnp.pad(x, ((0, _NPAD - _N), (0, 0)))
    b1r = b1.reshape(1, _D)
    b2r = b2.reshape(1, _D)

    deg_kernel, scatter_kernel = _get_sc_kernels()
    degp = deg_kernel(dst_flat).reshape(_NW, _NPAD)
    y1 = _t1(x_p, W1, degp)
    acc1 = scatter_kernel(y1, src_p, dst_p)
    y2 = _t2(acc1, y1, degp, b1r, W2)
    acc2 = scatter_kernel(y2, src_p, dst_p)
    out, logp = _t3(acc2, y2, degp, b2r)
    return out[:_N], logp[:_N]


# speedup vs baseline: 1.2884x; 1.0280x over previous
import functools

import jax
import jax.numpy as jnp
from jax import lax
from jax.experimental import pallas as pl
from jax.experimental.pallas import tpu as pltpu
from jax.experimental.pallas import tpu_sc as plsc

_N = 10000
_E = 320000
_D = 128
_NC, _NS = 2, 16
_NW = _NC * _NS
_NPAD = 10240
_CHUNK = 128
_NCHUNK = 2560
_BLK = 16
_CPT0 = 128
_CPT1 = _NCHUNK // _NS - _CPT0
_EPAD = _NCHUNK * _CHUNK
_EPT = _EPAD // _NW
_RPT = _NPAD // _NS
_NSUB = 2
_BM = 1024


def _deg_body(dst_hbm, out_hbm, dst_v, deg_v):
    cid = lax.axis_index("c")
    sid = lax.axis_index("s")
    wid = sid * _NC + cid
    pltpu.sync_copy(dst_hbm.at[wid], dst_v)

    def zero(i, carry):
        deg_v[i, :] = jnp.zeros((_NS,), jnp.float32)
        return carry

    lax.fori_loop(0, _RPT, zero, 0)

    ones = jnp.ones((16,), jnp.float32)

    def accum(i, carry):
        idx = dst_v[pl.ds(i * 16, 16)]
        plsc.addupdate_scatter(deg_v, [idx >> 4, idx & 15], ones)
        return carry

    lax.fori_loop(0, _EPT // 16, accum, 0)
    pltpu.sync_copy(deg_v, out_hbm.at[wid])


def _scatter_body(
    y_hbm, src_hbm, dst_hbm, out_hbm, src_ring, dst_ring, buf0, buf1, acc, sem0, sem1
):
    cid = lax.axis_index("c")
    sid = lax.axis_index("s")
    cpt = jnp.where(cid == 0, _CPT0, _CPT1)
    base = jnp.where(cid == 0, sid * _CPT0, _NS * _CPT0 + sid * _CPT1)

    zeros16 = jnp.zeros((16,), jnp.float32)

    def zrow(k, carry):
        buf0[k // 8, pl.ds((k % 8) * 16, 16)] = zeros16
        return carry

    lax.fori_loop(0, _CHUNK * 8, zrow, 0)
    for r in range(_RPT // _CHUNK):
        pltpu.sync_copy(buf0, acc.at[pl.ds(sid * _RPT + r * _CHUNK, _CHUNK)])

    _SUB = _CHUNK // _NSUB

    def gather_start(c, buf, sem):
        h = (c // _BLK) % 2
        s = c % _BLK
        for k in range(_NSUB):
            pltpu.async_copy(
                y_hbm.at[src_ring.at[h, s, pl.ds(k * _SUB, _SUB)]],
                buf.at[pl.ds(k * _SUB, _SUB)],
                sem,
            )

    def gather_wait(c, buf, sem):
        h = (c // _BLK) % 2
        s = c % _BLK
        for k in range(_NSUB):
            pltpu.make_async_copy(
                y_hbm.at[src_ring.at[h, s, pl.ds(k * _SUB, _SUB)]],
                buf.at[pl.ds(k * _SUB, _SUB)],
                sem,
            ).wait()

    pltpu.sync_copy(src_hbm.at[pl.ds(base, _BLK)], src_ring.at[0])
    pltpu.sync_copy(dst_hbm.at[pl.ds(base, _BLK)], dst_ring.at[0])
    gather_start(0, buf0, sem0)
    plsc.subcore_barrier()

    def body(g, carry):
        c0 = 2 * g
        c1 = c0 + 1
        c2 = c0 + 2

        @pl.when(jnp.logical_and(c0 % _BLK == 0, c0 + _BLK < cpt))
        def _():
            b1 = c0 // _BLK + 1
            pltpu.sync_copy(
                src_hbm.at[pl.ds(base + b1 * _BLK, _BLK)], src_ring.at[b1 % 2]
            )
            pltpu.sync_copy(
                dst_hbm.at[pl.ds(base + b1 * _BLK, _BLK)], dst_ring.at[b1 % 2]
            )

        gather_start(c1, buf1, sem1)
        gather_wait(c0, buf0, sem0)
        pltpu.sync_copy(
            buf0, acc.at[dst_ring.at[(c0 // _BLK) % 2, c0 % _BLK]], add=True
        )

        @pl.when(c2 < cpt)
        def _():
            gather_start(c2, buf0, sem0)

        gather_wait(c1, buf1, sem1)
        pltpu.sync_copy(
            buf1, acc.at[dst_ring.at[(c1 // _BLK) % 2, c1 % _BLK]], add=True
        )
        return carry

    lax.fori_loop(0, cpt // 2, body, 0)
    plsc.subcore_barrier()
    pltpu.sync_copy(
        acc.at[pl.ds(sid * _RPT, _RPT)], out_hbm.at[cid, pl.ds(sid * _RPT, _RPT)]
    )


@functools.cache
def _get_sc_kernels():
    mesh = plsc.VectorSubcoreMesh(
        core_axis_name="c", subcore_axis_name="s", num_cores=_NC, num_subcores=_NS
    )
    deg = pl.kernel(
        _deg_body,
        out_type=jax.ShapeDtypeStruct((_NW, _RPT, _NS), jnp.float32),
        mesh=mesh,
        scratch_types=[
            pltpu.VMEM((_EPT,), jnp.int32),
            pltpu.VMEM((_RPT, _NS), jnp.float32),
        ],
        compiler_params=pltpu.CompilerParams(needs_layout_passes=False),
    )
    scat = pl.kernel(
        _scatter_body,
        out_type=jax.ShapeDtypeStruct((_NC, _NPAD, _D), jnp.float32),
        mesh=mesh,
        scratch_types=[
            pltpu.VMEM((2, _BLK, _CHUNK), jnp.int32),
            pltpu.VMEM((2, _BLK, _CHUNK), jnp.int32),
            pltpu.VMEM((_CHUNK, _D), jnp.float32),
            pltpu.VMEM((_CHUNK, _D), jnp.float32),
            pltpu.VMEM_SHARED((_NPAD, _D), jnp.float32),
            pltpu.SemaphoreType.DMA,
            pltpu.SemaphoreType.DMA,
        ],
    )
    return deg, scat


def _dinv_block(deg_ref):
    deg = jnp.sum(deg_ref[...], axis=0) + 1.0
    return lax.rsqrt(deg)


def _t1a_body(x_ref, w_ref, o_ref):
    o_ref[...] = jnp.dot(x_ref[...], w_ref[...], preferred_element_type=jnp.float32)


def _t1b_body(xw_ref, deg_ref, o_ref):
    dinv = _dinv_block(deg_ref)
    o_ref[...] = xw_ref[...] * dinv[:, None]


def _t2_body(a_ref, y_ref, deg_ref, b_ref, w_ref, o_ref):
    i = pl.program_id(0)
    dinv = _dinv_block(deg_ref)
    pre = (a_ref[0] + a_ref[1] + y_ref[...]) * dinv[:, None] + b_ref[...]
    h = jnp.maximum(pre, 0.0)
    y2 = jnp.dot(h, w_ref[...], preferred_element_type=jnp.float32) * dinv[:, None]
    rows = i * _BM + lax.broadcasted_iota(jnp.int32, (_BM, 1), 0)
    o_ref[...] = jnp.where(rows < _N, y2, 0.0)


def _t3_body(a_ref, y_ref, deg_ref, b_ref, o_ref, lp_ref):
    dinv = _dinv_block(deg_ref)
    out = (a_ref[0] + a_ref[1] + y_ref[...]) * dinv[:, None] + b_ref[...]
    o_ref[...] = out
    m = jnp.max(out, axis=1, keepdims=True)
    ex = jnp.exp(out - m)
    lse = jnp.log(jnp.sum(ex, axis=1, keepdims=True)) + m
    lp_ref[...] = out - lse


_GRID = (_NPAD // _BM,)
_row_spec = pl.BlockSpec((_BM, _D), lambda i: (i, 0))
_acc_spec = pl.BlockSpec((_NC, _BM, _D), lambda i: (0, i, 0))
_deg_spec = pl.BlockSpec((_NW, _BM), lambda i: (0, i))
_w_spec = pl.BlockSpec((_D, _D), lambda i: (0, 0))
_b_spec = pl.BlockSpec((1, _D), lambda i: (0, 0))
_f32 = jnp.float32

_t1a = pl.pallas_call(
    _t1a_body,
    grid=_GRID,
    in_specs=[_row_spec, _w_spec],
    out_specs=_row_spec,
    out_shape=jax.ShapeDtypeStruct((_NPAD, _D), _f32),
)

_t1b = pl.pallas_call(
    _t1b_body,
    grid=_GRID,
    in_specs=[_row_spec, _deg_spec],
    out_specs=_row_spec,
    out_shape=jax.ShapeDtypeStruct((_NPAD, _D), _f32),
)

_t2 = pl.pallas_call(
    _t2_body,
    grid=_GRID,
    in_specs=[_acc_spec, _row_spec, _deg_spec, _b_spec, _w_spec],
    out_specs=_row_spec,
    out_shape=jax.ShapeDtypeStruct((_NPAD, _D), _f32),
)

_t3 = pl.pallas_call(
    _t3_body,
    grid=_GRID,
    in_specs=[_acc_spec, _row_spec, _deg_spec, _b_spec],
    out_specs=[_row_spec, _row_spec],
    out_shape=[
        jax.ShapeDtypeStruct((_NPAD, _D), _f32),
        jax.ShapeDtypeStruct((_NPAD, _D), _f32),
    ],
)


def kernel(x, edge_index, W1, b1, W2, b2):
    src = edge_index[0].astype(jnp.int32)
    dst = edge_index[1].astype(jnp.int32)
    pad = jnp.full((_EPAD - _E,), _N, jnp.int32)
    src_p = jnp.concatenate([src, pad]).reshape(_NCHUNK, _CHUNK)
    dst_p = jnp.concatenate([dst, pad]).reshape(_NCHUNK, _CHUNK)
    dst_flat = dst_p.reshape(_NW, _EPT)
    x_p = jnp.pad(x, ((0, _NPAD - _N), (0, 0)))
    b1r = b1.reshape(1, _D)
    b2r = b2.reshape(1, _D)

    deg_kernel, scatter_kernel = _get_sc_kernels()
    xw = _t1a(x_p, W1)
    degp = deg_kernel(dst_flat).reshape(_NW, _NPAD)
    y1 = _t1b(xw, degp)
    acc1 = scatter_kernel(y1, src_p, dst_p)
    y2 = _t2(acc1, y1, degp, b1r, W2)
    acc2 = scatter_kernel(y2, src_p, dst_p)
    out, logp = _t3(acc2, y2, degp, b2r)
    return out[:_N], logp[:_N]


# speedup vs baseline: 1.4193x; 1.1016x over previous
import functools

import jax
import jax.numpy as jnp
from jax import lax
from jax.experimental import pallas as pl
from jax.experimental.pallas import tpu as pltpu
from jax.experimental.pallas import tpu_sc as plsc

_N = 10000
_E = 320000
_D = 128
_NC, _NS = 2, 16
_NW = _NC * _NS
_NPAD = 10240
_CHUNK = 128
_NCHUNK = 2560
_BLK = 16
_CPT0 = 144
_CPT1 = _NCHUNK // _NS - _CPT0
_EPAD = _NCHUNK * _CHUNK
_EPT = _EPAD // _NW
_RPT = _NPAD // _NS
_NSUB = 2
_BM = 1024


def _deg_body(dst_hbm, out_hbm, dst_v, deg_v):
    cid = lax.axis_index("c")
    sid = lax.axis_index("s")
    wid = sid * _NC + cid
    pltpu.sync_copy(dst_hbm.at[wid], dst_v)

    def zero(i, carry):
        deg_v[i, :] = jnp.zeros((_NS,), jnp.float32)
        return carry

    lax.fori_loop(0, _RPT, zero, 0)

    ones = jnp.ones((16,), jnp.float32)

    def accum(i, carry):
        idx = dst_v[pl.ds(i * 16, 16)]
        plsc.addupdate_scatter(deg_v, [idx >> 4, idx & 15], ones)
        return carry

    lax.fori_loop(0, _EPT // 16, accum, 0)
    pltpu.sync_copy(deg_v, out_hbm.at[wid])


def _scatter_body(
    y_hbm, src_hbm, dst_hbm, out_hbm, src_ring, dst_ring, buf0, buf1, acc, sem0, sem1
):
    cid = lax.axis_index("c")
    sid = lax.axis_index("s")
    cpt = jnp.where(cid == 0, _CPT0, _CPT1)
    base = jnp.where(cid == 0, sid * _CPT0, _NS * _CPT0 + sid * _CPT1)

    zeros16 = jnp.zeros((16,), jnp.float32)

    def zrow(k, carry):
        buf0[k // 8, pl.ds((k % 8) * 16, 16)] = zeros16
        return carry

    lax.fori_loop(0, _CHUNK * 8, zrow, 0)
    for r in range(_RPT // _CHUNK):
        pltpu.sync_copy(buf0, acc.at[pl.ds(sid * _RPT + r * _CHUNK, _CHUNK)])

    _SUB = _CHUNK // _NSUB

    def gather_start(c, buf, sem):
        h = (c // _BLK) % 2
        s = c % _BLK
        for k in range(_NSUB):
            pltpu.async_copy(
                y_hbm.at[src_ring.at[h, s, pl.ds(k * _SUB, _SUB)]],
                buf.at[pl.ds(k * _SUB, _SUB)],
                sem,
            )

    def gather_wait(c, buf, sem):
        h = (c // _BLK) % 2
        s = c % _BLK
        for k in range(_NSUB):
            pltpu.make_async_copy(
                y_hbm.at[src_ring.at[h, s, pl.ds(k * _SUB, _SUB)]],
                buf.at[pl.ds(k * _SUB, _SUB)],
                sem,
            ).wait()

    pltpu.sync_copy(src_hbm.at[pl.ds(base, _BLK)], src_ring.at[0])
    pltpu.sync_copy(dst_hbm.at[pl.ds(base, _BLK)], dst_ring.at[0])
    gather_start(0, buf0, sem0)
    plsc.subcore_barrier()

    def body(g, carry):
        c0 = 2 * g
        c1 = c0 + 1
        c2 = c0 + 2

        @pl.when(jnp.logical_and(c0 % _BLK == 0, c0 + _BLK < cpt))
        def _():
            b1 = c0 // _BLK + 1
            pltpu.sync_copy(
                src_hbm.at[pl.ds(base + b1 * _BLK, _BLK)], src_ring.at[b1 % 2]
            )
            pltpu.sync_copy(
                dst_hbm.at[pl.ds(base + b1 * _BLK, _BLK)], dst_ring.at[b1 % 2]
            )

        gather_start(c1, buf1, sem1)
        gather_wait(c0, buf0, sem0)
        pltpu.sync_copy(
            buf0, acc.at[dst_ring.at[(c0 // _BLK) % 2, c0 % _BLK]], add=True
        )

        @pl.when(c2 < cpt)
        def _():
            gather_start(c2, buf0, sem0)

        gather_wait(c1, buf1, sem1)
        pltpu.sync_copy(
            buf1, acc.at[dst_ring.at[(c1 // _BLK) % 2, c1 % _BLK]], add=True
        )
        return carry

    lax.fori_loop(0, cpt // 2, body, 0)
    plsc.subcore_barrier()
    pltpu.sync_copy(
        acc.at[pl.ds(sid * _RPT, _RPT)], out_hbm.at[cid, pl.ds(sid * _RPT, _RPT)]
    )


@functools.cache
def _get_sc_kernels():
    mesh = plsc.VectorSubcoreMesh(
        core_axis_name="c", subcore_axis_name="s", num_cores=_NC, num_subcores=_NS
    )
    deg = pl.kernel(
        _deg_body,
        out_type=jax.ShapeDtypeStruct((_NW, _RPT, _NS), jnp.float32),
        mesh=mesh,
        scratch_types=[
            pltpu.VMEM((_EPT,), jnp.int32),
            pltpu.VMEM((_RPT, _NS), jnp.float32),
        ],
        compiler_params=pltpu.CompilerParams(needs_layout_passes=False),
    )
    scat = pl.kernel(
        _scatter_body,
        out_type=jax.ShapeDtypeStruct((_NC, _NPAD, _D), jnp.float32),
        mesh=mesh,
        scratch_types=[
            pltpu.VMEM((2, _BLK, _CHUNK), jnp.int32),
            pltpu.VMEM((2, _BLK, _CHUNK), jnp.int32),
            pltpu.VMEM((_CHUNK, _D), jnp.float32),
            pltpu.VMEM((_CHUNK, _D), jnp.float32),
            pltpu.VMEM_SHARED((_NPAD, _D), jnp.float32),
            pltpu.SemaphoreType.DMA,
            pltpu.SemaphoreType.DMA,
        ],
    )
    return deg, scat


def _dinv_block(deg_ref):
    deg = jnp.sum(deg_ref[...], axis=0) + 1.0
    return lax.rsqrt(deg)


def _t1a_body(x_ref, w_ref, o_ref):
    o_ref[...] = jnp.dot(x_ref[...], w_ref[...], preferred_element_type=jnp.float32)


def _t1b_body(xw_ref, deg_ref, o_ref):
    dinv = _dinv_block(deg_ref)
    o_ref[...] = xw_ref[...] * dinv[:, None]


def _t2_body(a_ref, y_ref, deg_ref, b_ref, w_ref, o_ref):
    i = pl.program_id(0)
    dinv = _dinv_block(deg_ref)
    pre = (a_ref[0] + a_ref[1] + y_ref[...]) * dinv[:, None] + b_ref[...]
    h = jnp.maximum(pre, 0.0)
    y2 = jnp.dot(h, w_ref[...], preferred_element_type=jnp.float32) * dinv[:, None]
    rows = i * _BM + lax.broadcasted_iota(jnp.int32, (_BM, 1), 0)
    o_ref[...] = jnp.where(rows < _N, y2, 0.0)


def _t3_body(a_ref, y_ref, deg_ref, b_ref, o_ref, lp_ref):
    dinv = _dinv_block(deg_ref)
    out = (a_ref[0] + a_ref[1] + y_ref[...]) * dinv[:, None] + b_ref[...]
    o_ref[...] = out
    m = jnp.max(out, axis=1, keepdims=True)
    ex = jnp.exp(out - m)
    lse = jnp.log(jnp.sum(ex, axis=1, keepdims=True)) + m
    lp_ref[...] = out - lse


_GRID = (_NPAD // _BM,)
_row_spec = pl.BlockSpec((_BM, _D), lambda i: (i, 0))
_acc_spec = pl.BlockSpec((_NC, _BM, _D), lambda i: (0, i, 0))
_deg_spec = pl.BlockSpec((_NW, _BM), lambda i: (0, i))
_w_spec = pl.BlockSpec((_D, _D), lambda i: (0, 0))
_b_spec = pl.BlockSpec((1, _D), lambda i: (0, 0))
_f32 = jnp.float32

_t1a = pl.pallas_call(
    _t1a_body,
    grid=_GRID,
    in_specs=[_row_spec, _w_spec],
    out_specs=_row_spec,
    out_shape=jax.ShapeDtypeStruct((_NPAD, _D), _f32),
)

_t1b = pl.pallas_call(
    _t1b_body,
    grid=_GRID,
    in_specs=[_row_spec, _deg_spec],
    out_specs=_row_spec,
    out_shape=jax.ShapeDtypeStruct((_NPAD, _D), _f32),
)

_t2 = pl.pallas_call(
    _t2_body,
    grid=_GRID,
    in_specs=[_acc_spec, _row_spec, _deg_spec, _b_spec, _w_spec],
    out_specs=_row_spec,
    out_shape=jax.ShapeDtypeStruct((_NPAD, _D), _f32),
)

_t3 = pl.pallas_call(
    _t3_body,
    grid=_GRID,
    in_specs=[_acc_spec, _row_spec, _deg_spec, _b_spec],
    out_specs=[_row_spec, _row_spec],
    out_shape=[
        jax.ShapeDtypeStruct((_NPAD, _D), _f32),
        jax.ShapeDtypeStruct((_NPAD, _D), _f32),
    ],
)


def kernel(x, edge_index, W1, b1, W2, b2):
    src = edge_index[0].astype(jnp.int32)
    dst = edge_index[1].astype(jnp.int32)
    pad = jnp.full((_EPAD - _E,), _N, jnp.int32)
    src_p = jnp.concatenate([src, pad]).reshape(_NCHUNK, _CHUNK)
    dst_p = jnp.concatenate([dst, pad]).reshape(_NCHUNK, _CHUNK)
    dst_flat = dst_p.reshape(_NW, _EPT)
    x_p = jnp.pad(x, ((0, _NPAD - _N), (0, 0)))
    b1r = b1.reshape(1, _D)
    b2r = b2.reshape(1, _D)

    deg_kernel, scatter_kernel = _get_sc_kernels()
    xw = _t1a(x_p, W1)
    degp = deg_kernel(dst_flat).reshape(_NW, _NPAD)
    y1 = _t1b(xw, degp)
    acc1 = scatter_kernel(y1, src_p, dst_p)
    y2 = _t2(acc1, y1, degp, b1r, W2)
    acc2 = scatter_kernel(y2, src_p, dst_p)
    out, logp = _t3(acc2, y2, degp, b2r)
    return out[:_N], logp[:_N]


# speedup vs baseline: 1.4317x; 1.0088x over previous
import functools

import jax
import jax.numpy as jnp
from jax import lax
from jax.experimental import pallas as pl
from jax.experimental.pallas import tpu as pltpu
from jax.experimental.pallas import tpu_sc as plsc

_N = 10000
_E = 320000
_D = 128
_NC, _NS = 2, 16
_NW = _NC * _NS
_NPAD = 10240
_CHUNK = 128
_NCHUNK = 2560
_BLK = 16
_CPT0 = 152
_CPT1 = _NCHUNK // _NS - _CPT0
_EPAD = _NCHUNK * _CHUNK
_EPT = _EPAD // _NW
_RPT = _NPAD // _NS
_NSUB = 2
_BM = 1024


def _deg_body(dst_hbm, out_hbm, dst_v, deg_v):
    cid = lax.axis_index("c")
    sid = lax.axis_index("s")
    wid = sid * _NC + cid
    pltpu.sync_copy(dst_hbm.at[wid], dst_v)

    def zero(i, carry):
        deg_v[i, :] = jnp.zeros((_NS,), jnp.float32)
        return carry

    lax.fori_loop(0, _RPT, zero, 0)

    ones = jnp.ones((16,), jnp.float32)

    def accum(i, carry):
        idx = dst_v[pl.ds(i * 16, 16)]
        plsc.addupdate_scatter(deg_v, [idx >> 4, idx & 15], ones)
        return carry

    lax.fori_loop(0, _EPT // 16, accum, 0)
    pltpu.sync_copy(deg_v, out_hbm.at[wid])


def _scatter_body(
    y_hbm, src_hbm, dst_hbm, out_hbm, src_ring, dst_ring, buf0, buf1, acc, sem0, sem1
):
    cid = lax.axis_index("c")
    sid = lax.axis_index("s")
    cpt = jnp.where(cid == 0, _CPT0, _CPT1)
    base = jnp.where(cid == 0, sid * _CPT0, _NS * _CPT0 + sid * _CPT1)

    zeros16 = jnp.zeros((16,), jnp.float32)

    def zrow(k, carry):
        buf0[k // 8, pl.ds((k % 8) * 16, 16)] = zeros16
        return carry

    lax.fori_loop(0, _CHUNK * 8, zrow, 0)
    for r in range(_RPT // _CHUNK):
        pltpu.sync_copy(buf0, acc.at[pl.ds(sid * _RPT + r * _CHUNK, _CHUNK)])

    _SUB = _CHUNK // _NSUB

    def gather_start(c, buf, sem):
        h = (c // _BLK) % 2
        s = c % _BLK
        for k in range(_NSUB):
            pltpu.async_copy(
                y_hbm.at[src_ring.at[h, s, pl.ds(k * _SUB, _SUB)]],
                buf.at[pl.ds(k * _SUB, _SUB)],
                sem,
            )

    def gather_wait(c, buf, sem):
        h = (c // _BLK) % 2
        s = c % _BLK
        for k in range(_NSUB):
            pltpu.make_async_copy(
                y_hbm.at[src_ring.at[h, s, pl.ds(k * _SUB, _SUB)]],
                buf.at[pl.ds(k * _SUB, _SUB)],
                sem,
            ).wait()

    pltpu.sync_copy(src_hbm.at[pl.ds(base, _BLK)], src_ring.at[0])
    pltpu.sync_copy(dst_hbm.at[pl.ds(base, _BLK)], dst_ring.at[0])
    gather_start(0, buf0, sem0)
    plsc.subcore_barrier()

    def body(g, carry):
        c0 = 2 * g
        c1 = c0 + 1
        c2 = c0 + 2

        @pl.when(jnp.logical_and(c0 % _BLK == 0, c0 + _BLK < cpt))
        def _():
            b1 = c0 // _BLK + 1
            pltpu.sync_copy(
                src_hbm.at[pl.ds(base + b1 * _BLK, _BLK)], src_ring.at[b1 % 2]
            )
            pltpu.sync_copy(
                dst_hbm.at[pl.ds(base + b1 * _BLK, _BLK)], dst_ring.at[b1 % 2]
            )

        gather_start(c1, buf1, sem1)
        gather_wait(c0, buf0, sem0)
        pltpu.sync_copy(
            buf0, acc.at[dst_ring.at[(c0 // _BLK) % 2, c0 % _BLK]], add=True
        )

        @pl.when(c2 < cpt)
        def _():
            gather_start(c2, buf0, sem0)

        gather_wait(c1, buf1, sem1)
        pltpu.sync_copy(
            buf1, acc.at[dst_ring.at[(c1 // _BLK) % 2, c1 % _BLK]], add=True
        )
        return carry

    lax.fori_loop(0, cpt // 2, body, 0)
    plsc.subcore_barrier()
    pltpu.sync_copy(
        acc.at[pl.ds(sid * _RPT, _RPT)], out_hbm.at[cid, pl.ds(sid * _RPT, _RPT)]
    )


@functools.cache
def _get_sc_kernels():
    mesh = plsc.VectorSubcoreMesh(
        core_axis_name="c", subcore_axis_name="s", num_cores=_NC, num_subcores=_NS
    )
    deg = pl.kernel(
        _deg_body,
        out_type=jax.ShapeDtypeStruct((_NW, _RPT, _NS), jnp.float32),
        mesh=mesh,
        scratch_types=[
            pltpu.VMEM((_EPT,), jnp.int32),
            pltpu.VMEM((_RPT, _NS), jnp.float32),
        ],
        compiler_params=pltpu.CompilerParams(needs_layout_passes=False),
    )
    scat = pl.kernel(
        _scatter_body,
        out_type=jax.ShapeDtypeStruct((_NC, _NPAD, _D), jnp.float32),
        mesh=mesh,
        scratch_types=[
            pltpu.VMEM((2, _BLK, _CHUNK), jnp.int32),
            pltpu.VMEM((2, _BLK, _CHUNK), jnp.int32),
            pltpu.VMEM((_CHUNK, _D), jnp.float32),
            pltpu.VMEM((_CHUNK, _D), jnp.float32),
            pltpu.VMEM_SHARED((_NPAD, _D), jnp.float32),
            pltpu.SemaphoreType.DMA,
            pltpu.SemaphoreType.DMA,
        ],
    )
    return deg, scat


def _dinv_block(deg_ref):
    deg = jnp.sum(deg_ref[...], axis=0) + 1.0
    return lax.rsqrt(deg)


def _t1a_body(x_ref, w_ref, o_ref):
    o_ref[...] = jnp.dot(x_ref[...], w_ref[...], preferred_element_type=jnp.float32)


def _t1b_body(xw_ref, deg_ref, o_ref):
    dinv = _dinv_block(deg_ref)
    o_ref[...] = xw_ref[...] * dinv[:, None]


def _t2_body(a_ref, y_ref, deg_ref, b_ref, w_ref, o_ref):
    i = pl.program_id(0)
    dinv = _dinv_block(deg_ref)
    pre = (a_ref[0] + a_ref[1] + y_ref[...]) * dinv[:, None] + b_ref[...]
    h = jnp.maximum(pre, 0.0)
    y2 = jnp.dot(h, w_ref[...], preferred_element_type=jnp.float32) * dinv[:, None]
    rows = i * _BM + lax.broadcasted_iota(jnp.int32, (_BM, 1), 0)
    o_ref[...] = jnp.where(rows < _N, y2, 0.0)


def _t3_body(a_ref, y_ref, deg_ref, b_ref, o_ref, lp_ref):
    dinv = _dinv_block(deg_ref)
    out = (a_ref[0] + a_ref[1] + y_ref[...]) * dinv[:, None] + b_ref[...]
    o_ref[...] = out
    m = jnp.max(out, axis=1, keepdims=True)
    ex = jnp.exp(out - m)
    lse = jnp.log(jnp.sum(ex, axis=1, keepdims=True)) + m
    lp_ref[...] = out - lse


_GRID = (_NPAD // _BM,)
_row_spec = pl.BlockSpec((_BM, _D), lambda i: (i, 0))
_acc_spec = pl.BlockSpec((_NC, _BM, _D), lambda i: (0, i, 0))
_deg_spec = pl.BlockSpec((_NW, _BM), lambda i: (0, i))
_w_spec = pl.BlockSpec((_D, _D), lambda i: (0, 0))
_b_spec = pl.BlockSpec((1, _D), lambda i: (0, 0))
_f32 = jnp.float32

_t1a = pl.pallas_call(
    _t1a_body,
    grid=_GRID,
    in_specs=[_row_spec, _w_spec],
    out_specs=_row_spec,
    out_shape=jax.ShapeDtypeStruct((_NPAD, _D), _f32),
)

_t1b = pl.pallas_call(
    _t1b_body,
    grid=_GRID,
    in_specs=[_row_spec, _deg_spec],
    out_specs=_row_spec,
    out_shape=jax.ShapeDtypeStruct((_NPAD, _D), _f32),
)

_t2 = pl.pallas_call(
    _t2_body,
    grid=_GRID,
    in_specs=[_acc_spec, _row_spec, _deg_spec, _b_spec, _w_spec],
    out_specs=_row_spec,
    out_shape=jax.ShapeDtypeStruct((_NPAD, _D), _f32),
)

_t3 = pl.pallas_call(
    _t3_body,
    grid=_GRID,
    in_specs=[_acc_spec, _row_spec, _deg_spec, _b_spec],
    out_specs=[_row_spec, _row_spec],
    out_shape=[
        jax.ShapeDtypeStruct((_NPAD, _D), _f32),
        jax.ShapeDtypeStruct((_NPAD, _D), _f32),
    ],
)


def kernel(x, edge_index, W1, b1, W2, b2):
    src = edge_index[0].astype(jnp.int32)
    dst = edge_index[1].astype(jnp.int32)
    pad = jnp.full((_EPAD - _E,), _N, jnp.int32)
    src_p = jnp.concatenate([src, pad]).reshape(_NCHUNK, _CHUNK)
    dst_p = jnp.concatenate([dst, pad]).reshape(_NCHUNK, _CHUNK)
    dst_flat = dst_p.reshape(_NW, _EPT)
    x_p = jnp.pad(x, ((0, _NPAD - _N), (0, 0)))
    b1r = b1.reshape(1, _D)
    b2r = b2.reshape(1, _D)

    deg_kernel, scatter_kernel = _get_sc_kernels()
    xw = _t1a(x_p, W1)
    degp = deg_kernel(dst_flat).reshape(_NW, _NPAD)
    y1 = _t1b(xw, degp)
    acc1 = scatter_kernel(y1, src_p, dst_p)
    y2 = _t2(acc1, y1, degp, b1r, W2)
    acc2 = scatter_kernel(y2, src_p, dst_p)
    out, logp = _t3(acc2, y2, degp, b2r)
    return out[:_N], logp[:_N]
